# baseline (speedup 1.0000x reference)
import functools

import jax
import jax.numpy as jnp
from jax import lax
from jax.experimental import pallas as pl
from jax.experimental.pallas import tpu as pltpu
from jax.experimental.pallas import tpu_sc as plsc

ROWS, COLS = 4096, 50
D = 768
V = 64
NC, NS = 2, 16
NW = NC * NS
R_PER_W = ROWS // NW
NBUF = 2
GSTEPS = 8
NGROUPS = R_PER_W // GSTEPS
NLANE = 16

_mesh = plsc.VectorSubcoreMesh(core_axis_name="c", subcore_axis_name="s")


@functools.partial(
    pl.kernel,
    mesh=_mesh,
    out_type=jax.ShapeDtypeStruct((ROWS, COLS, D), jnp.float32),
    scratch_types=[
        pltpu.VMEM((R_PER_W, COLS), jnp.int32),
        pltpu.VMEM((NGROUPS, NLANE), jnp.int32),
        pltpu.VMEM((NBUF, COLS, D), jnp.float32),
        pltpu.VMEM((NBUF, NLANE, D), jnp.float32),
        pltpu.SemaphoreType.DMA(NBUF),
        pltpu.SemaphoreType.DMA(NBUF),
        pltpu.SemaphoreType.DMA(NBUF),
    ],
)
def _embed(table_hbm, idx_hbm, tails_hbm, out_hbm, idx_v, tails_v, ring, fix,
           sem_g, sem_s, sem_f):
    cid = lax.axis_index("c")
    sid = lax.axis_index("s")
    wid = sid * NC + cid
    base = wid * R_PER_W

    pltpu.sync_copy(idx_hbm.at[pl.ds(base, R_PER_W)], idx_v)
    pltpu.sync_copy(tails_hbm.at[wid], tails_v)

    def gather(n, b):
        pltpu.async_copy(table_hbm.at[idx_v.at[n]], ring.at[b], sem_g.at[b])

    def gather_wait(n, b):
        pltpu.make_async_copy(table_hbm.at[idx_v.at[n]], ring.at[b],
                              sem_g.at[b]).wait()

    def fixgather(m, f):
        pltpu.async_copy(table_hbm.at[tails_v.at[m]], fix.at[f], sem_f.at[f])

    def fixwait(m, f):
        pltpu.make_async_copy(table_hbm.at[tails_v.at[m]], fix.at[f],
                              sem_f.at[f]).wait()

    def store(n, b):
        pltpu.async_copy(ring.at[b], out_hbm.at[base + n], sem_s.at[b])

    def store_wait(n, b):
        pltpu.make_async_copy(ring.at[b], out_hbm.at[base + n],
                              sem_s.at[b]).wait()

    def step(n, b, k, fm, wait_old_store, prefetch):
        gather_wait(n, b)
        for r in range(2):
            for c in range(D // NLANE):
                ring[b, COLS - 2 + r, pl.ds(c * NLANE, NLANE)] = (
                    fix[fm, 2 * k + r, pl.ds(c * NLANE, NLANE)])
        store(n, b)
        if wait_old_store:
            store_wait(n - 1, 1 - b)
        if prefetch:
            gather(n + 1, 1 - b)

    def group(m, fm, first=False, last=False):
        fixwait(m, fm)
        if not last:
            fixgather(m + 1, 1 - fm)
        for k in range(GSTEPS):
            n = m * GSTEPS + k
            step(n, k % 2, k, fm,
                 wait_old_store=not (first and k == 0),
                 prefetch=not (last and k == GSTEPS - 1))

    gather(0, 0)
    fixgather(0, 0)

    group(0, 0, first=True)

    def body(q, carry):
        group(2 * q + 1, 1)
        group(2 * q + 2, 0)
        return carry

    lax.fori_loop(0, NGROUPS // 2 - 1, body, 0)

    group(NGROUPS - 1, 1, last=True)

    store_wait(R_PER_W - 1, 1)


def kernel(indices, table):
    idx = indices.astype(jnp.int32)
    tails = idx[:, COLS - 2:].reshape(NW, NGROUPS, NLANE)
    return _embed(table, idx, tails)
